# baseline (speedup 1.0000x reference)
import functools

import jax
import jax.numpy as jnp
from jax import lax
from jax.experimental import pallas as pl
from jax.experimental.pallas import tpu as pltpu
from jax.experimental.pallas import tpu_sc as plsc

NN = 100000
EE = 1600000
HH = 32
HF = 16
GG = 256
OC = 45

NSC = 2
TPS = 16
CH = 80
GRP = 8
ECH = EE // CH
EG = ECH // GRP
DG = EG // NSC
MCH = 125
MGRP = 4
MECH = EE // MCH
MEG = MECH // MGRP
MGT = MEG // TPS
NCH = NN // CH
NG = (NCH + GRP - 1) // GRP
ZR = 400
NZG = NN // ZR

_MESH = plsc.VectorSubcoreMesh(
    core_axis_name="c", subcore_axis_name="s", num_cores=NSC, num_subcores=TPS)

_f32 = jnp.float32
_i32 = jnp.int32


def _fill_rows(ref, nrows, value):
    v = jnp.full((16,), value, _f32)

    def body(i, _):
        ref[i, :] = v
        return 0

    lax.fori_loop(0, nrows, body, 0)


def _fill_flat(ref, n, value):
    v = jnp.full((16,), value, _f32)

    def body(i, _):
        ref[pl.ds(i * 16, 16)] = v
        return 0

    lax.fori_loop(0, n // 16, body, 0)


@functools.partial(
    pl.kernel,
    out_type=jax.ShapeDtypeStruct((NSC, NN), _f32),
    mesh=_MESH,
    compiler_params=pltpu.CompilerParams(use_tc_tiling_on_sc=False),
    scratch_types=[
        pltpu.VMEM((GRP, CH), _i32),
        pltpu.VMEM((CH,), _f32),
        pltpu.VMEM((ZR,), _f32),
        pltpu.VMEM_SHARED((NN,), _f32),
    ],
)
def _deg_kernel(dst_hbm, deg_out, dbuf, ones, zb, dacc):
    c = lax.axis_index("c")
    t = lax.axis_index("s")

    _fill_flat(zb, ZR, 0.0)
    _fill_flat(ones, CH, 1.0)

    def z_body(i, _):
        zg = t + TPS * i

        @pl.when(zg < NZG)
        def _():
            pltpu.sync_copy(zb, dacc.at[pl.ds(zg * ZR, ZR)])

        return 0

    lax.fori_loop(0, (NZG + TPS - 1) // TPS, z_body, 0)

    plsc.subcore_barrier()

    def blk(i, _):
        g = t + TPS * i

        @pl.when(g < DG)
        def _():
            pltpu.sync_copy(dst_hbm.at[pl.ds((c * DG + g) * GRP, GRP), :], dbuf)

            def inner(j, _):
                pltpu.sync_copy(ones, dacc.at[dbuf.at[j]], add=True)
                return 0

            lax.fori_loop(0, GRP, inner, 0)

        return 0

    lax.fori_loop(0, (DG + TPS - 1) // TPS, blk, 0)

    plsc.subcore_barrier()

    @pl.when(t == 0)
    def _():
        pltpu.sync_copy(dacc, deg_out.at[c])


@functools.partial(
    pl.kernel,
    out_type=jax.ShapeDtypeStruct((NSC, NN, HF), _f32),
    mesh=_MESH,
    compiler_params=pltpu.CompilerParams(use_tc_tiling_on_sc=False),
    scratch_types=[
        pltpu.VMEM((2, MGRP, MCH), _i32),
        pltpu.VMEM((4, MGRP, MCH), _i32),
        pltpu.VMEM((2, MGRP, MCH, HF), _f32),
        pltpu.VMEM((ZR, 16), _f32),
        pltpu.VMEM_SHARED((NN, HF), _f32),
        pltpu.SemaphoreType.DMA((2,)),
        pltpu.SemaphoreType.DMA((4,)),
        pltpu.SemaphoreType.DMA((2,)),
        pltpu.SemaphoreType.DMA((2,)),
    ],
)
def _mp_kernel(ys_hbm, src_hbm, dst_hbm, s_out, sbuf, dbuf, rows, zb, acc,
               issem, idsem, gsem, ssem):
    c = lax.axis_index("c")
    t = lax.axis_index("s")

    _fill_rows(zb, ZR, 0.0)

    def z_body(i, _):
        zg = t + TPS * i

        @pl.when(zg < NZG)
        def _():
            pltpu.sync_copy(zb, acc.at[pl.ds(zg * ZR, ZR), :])

        return 0

    lax.fori_loop(0, (NZG + TPS - 1) // TPS, z_body, 0)

    plsc.subcore_barrier()

    def ld_src(i):
        g = t * MGT + i
        return pltpu.make_async_copy(
            src_hbm.at[c, pl.ds(g * MGRP, MGRP), :],
            sbuf.at[lax.rem(i, 2)], issem.at[lax.rem(i, 2)])

    def ld_dst(i):
        g = t * MGT + i
        return pltpu.make_async_copy(
            dst_hbm.at[pl.ds(g * MGRP, MGRP), :],
            dbuf.at[lax.rem(i, 4)], idsem.at[lax.rem(i, 4)])

    def gath(i, j):
        p = lax.rem(i, 2)
        return pltpu.make_async_copy(
            ys_hbm.at[sbuf.at[p, j]], rows.at[p, j], gsem.at[p])

    def scat(i, j):
        p = lax.rem(i, 2)
        return pltpu.make_async_copy(
            rows.at[p, j], acc.at[dbuf.at[lax.rem(i, 4), j]], ssem.at[p])

    ld_src(0).start()
    ld_dst(0).start()
    ld_src(1).start()
    ld_dst(1).start()
    ld_src(0).wait()

    def issue_gathers(i):
        def body(j, _):
            gath(i, j).start()
            return 0

        lax.fori_loop(0, MGRP, body, 0)

    def wait_scatters(i):
        def body(j, _):
            scat(i, j).wait()
            return 0

        lax.fori_loop(0, MGRP, body, 0)

    issue_gathers(0)

    def step(i, _):
        @pl.when(i >= 1)
        def _():
            wait_scatters(i - 1)

        ld_dst(i).wait()

        def chunk(j, _):
            gath(i, j).wait()
            scat(i, j).start(add=True)
            return 0

        lax.fori_loop(0, MGRP, chunk, 0)

        @pl.when(i + 2 < MGT)
        def _():
            ld_src(i + 2).start()
            ld_dst(i + 2).start()

        @pl.when(i + 1 < MGT)
        def _():
            ld_src(i + 1).wait()
            issue_gathers(i + 1)

        return 0

    lax.fori_loop(0, MGT, step, 0)
    wait_scatters(MGT - 1)

    plsc.subcore_barrier()

    def w_body(i, _):
        zg = t + TPS * i

        @pl.when(zg < NZG)
        def _():
            pltpu.sync_copy(acc.at[pl.ds(zg * ZR, ZR), :],
                            s_out.at[c, pl.ds(zg * ZR, ZR), :])

        return 0

    lax.fori_loop(0, (NZG + TPS - 1) // TPS, w_body, 0)


@functools.partial(
    pl.kernel,
    out_type=(jax.ShapeDtypeStruct((NSC, GG, HF), _f32),
              jax.ShapeDtypeStruct((GG,), _f32)),
    mesh=_MESH,
    compiler_params=pltpu.CompilerParams(use_tc_tiling_on_sc=False),
    scratch_types=[
        pltpu.VMEM((GRP, CH), _i32),
        pltpu.VMEM((CH, HF), _f32),
        pltpu.VMEM((CH,), _f32),
        pltpu.VMEM((GG, 16), _f32),
        pltpu.VMEM((GG,), _f32),
        pltpu.VMEM_SHARED((GG, HF), _f32),
        pltpu.VMEM_SHARED((GG,), _f32),
    ],
)
def _pool_kernel(h3_hbm, batch_hbm, sums_out, cnt_out,
                 ibuf, rbuf, ones, zb, zb1, pacc, cacc):
    c = lax.axis_index("c")
    t = lax.axis_index("s")

    _fill_rows(zb, GG, 0.0)
    _fill_flat(zb1, GG, 0.0)
    _fill_flat(ones, CH, 1.0)

    @pl.when(t == 0)
    def _():
        pltpu.sync_copy(zb, pacc)
        pltpu.sync_copy(zb1, cacc)

    plsc.subcore_barrier()

    def body(i, _):
        g = t + TPS * i

        @pl.when(g < NG)
        def _():
            pltpu.sync_copy(batch_hbm.at[pl.ds(g * GRP, GRP), :], ibuf)

            def inner(j, _):
                cc = g * GRP + j

                @pl.when(cc < NCH)
                def _():
                    pltpu.sync_copy(h3_hbm.at[c, pl.ds(cc * CH, CH), :], rbuf)
                    pltpu.sync_copy(rbuf, pacc.at[ibuf.at[j]], add=True)

                    @pl.when(c == 0)
                    def _():
                        pltpu.sync_copy(ones, cacc.at[ibuf.at[j]], add=True)

                return 0

            lax.fori_loop(0, GRP, inner, 0)

        return 0

    lax.fori_loop(0, (NG + TPS - 1) // TPS, body, 0)

    plsc.subcore_barrier()

    @pl.when(t == 0)
    def _():
        pltpu.sync_copy(pacc, sums_out.at[c])

        @pl.when(c == 0)
        def _():
            pltpu.sync_copy(cacc, cnt_out)


NB = 4000


def _tc_pre_body(x_ref, deg_ref, w_ref, y_ref, dinv_ref):
    d = deg_ref[0] + deg_ref[1] + 1.0
    dinv = lax.rsqrt(d)
    xw = jnp.dot(x_ref[...], w_ref[...], preferred_element_type=_f32)
    y = xw * dinv
    y_ref[0] = y[:, :HF]
    y_ref[1] = y[:, HF:]
    dinv_ref[...] = dinv


def _tc_mid_body(s_ref, y_ref, dinv_ref, b_ref, w_ref, y2_ref):
    dinv = dinv_ref[...]
    b = b_ref[...]
    h_lo = jnp.maximum(dinv * (s_ref[0] + y_ref[0]) + b[:, :HF], 0.0)
    h_hi = jnp.maximum(dinv * (s_ref[1] + y_ref[1]) + b[:, HF:], 0.0)
    w = w_ref[...]
    y2 = (jnp.dot(h_lo, w[:HF, :], preferred_element_type=_f32)
          + jnp.dot(h_hi, w[HF:, :], preferred_element_type=_f32)) * dinv
    y2_ref[0] = y2[:, :HF]
    y2_ref[1] = y2[:, HF:]


def _tc_post_body(s_ref, y_ref, dinv_ref, b_ref, h3_ref):
    dinv = dinv_ref[...]
    b = b_ref[...]
    h3_ref[0] = dinv * (s_ref[0] + y_ref[0]) + b[:, :HF]
    h3_ref[1] = dinv * (s_ref[1] + y_ref[1]) + b[:, HF:]


def _tc_head_body(sums_ref, cnt_ref, wl_ref, bl_ref, out_ref):
    cnt = jnp.maximum(cnt_ref[...], 1.0)
    p_lo = sums_ref[0] / cnt
    p_hi = sums_ref[1] / cnt
    wl = wl_ref[...]
    out_ref[...] = (jnp.dot(p_lo, wl[:HF, :], preferred_element_type=_f32)
                    + jnp.dot(p_hi, wl[HF:, :], preferred_element_type=_f32)
                    + bl_ref[...])


_GRID = NN // NB

_spec_half = pl.BlockSpec((NSC, NB, HF), lambda i: (0, i, 0))
_spec_dinv = pl.BlockSpec((NB, 1), lambda i: (i, 0))


def _tc_pre(x, deg2, w1):
    return pl.pallas_call(
        _tc_pre_body,
        grid=(_GRID,),
        in_specs=[
            pl.BlockSpec((NB, 3), lambda i: (i, 0)),
            pl.BlockSpec((NSC, NB, 1), lambda i: (0, i, 0)),
            pl.BlockSpec((3, HH), lambda i: (0, 0)),
        ],
        out_specs=[_spec_half, _spec_dinv],
        out_shape=[jax.ShapeDtypeStruct((NSC, NN, HF), _f32),
                   jax.ShapeDtypeStruct((NN, 1), _f32)],
    )(x, deg2, w1)


def _tc_mid(s, y, dinv, b, w):
    return pl.pallas_call(
        _tc_mid_body,
        grid=(_GRID,),
        in_specs=[
            _spec_half, _spec_half, _spec_dinv,
            pl.BlockSpec((1, HH), lambda i: (0, 0)),
            pl.BlockSpec((HH, HH), lambda i: (0, 0)),
        ],
        out_specs=_spec_half,
        out_shape=jax.ShapeDtypeStruct((NSC, NN, HF), _f32),
    )(s, y, dinv, b, w)


def _tc_post(s, y, dinv, b):
    return pl.pallas_call(
        _tc_post_body,
        grid=(_GRID,),
        in_specs=[
            _spec_half, _spec_half, _spec_dinv,
            pl.BlockSpec((1, HH), lambda i: (0, 0)),
        ],
        out_specs=_spec_half,
        out_shape=jax.ShapeDtypeStruct((NSC, NN, HF), _f32),
    )(s, y, dinv, b)


def _tc_head(sums, cnt, wl, bl):
    return pl.pallas_call(
        _tc_head_body,
        in_specs=[
            pl.BlockSpec((NSC, GG, HF), lambda: (0, 0, 0)),
            pl.BlockSpec((GG, 1), lambda: (0, 0)),
            pl.BlockSpec((HH, OC), lambda: (0, 0)),
            pl.BlockSpec((1, OC), lambda: (0, 0)),
        ],
        out_specs=pl.BlockSpec((GG, OC), lambda: (0, 0)),
        out_shape=jax.ShapeDtypeStruct((GG, OC), _f32),
    )(sums, cnt, wl, bl)


def kernel(x, edge_index, batch, W1, b1, W2, b2, W3, b3, Wl, bl):
    src = edge_index[0]
    dst = edge_index[1]
    src2 = jnp.stack([src, src + NN]).reshape(NSC, MECH, MCH)
    dst2m = dst.reshape(MECH, MCH)
    dst2 = dst.reshape(ECH, CH)
    batch2 = jnp.pad(batch.reshape(NCH, CH), ((0, NG * GRP - NCH), (0, 0)))

    deg2 = _deg_kernel(dst2)
    deg2r = deg2.reshape(NSC, NN, 1)

    y1, dinv = _tc_pre(x, deg2r, W1)
    s1 = _mp_kernel(y1.reshape(NSC * NN, HF), src2, dst2m)
    y2 = _tc_mid(s1, y1, dinv, b1.reshape(1, HH), W2)
    s2 = _mp_kernel(y2.reshape(NSC * NN, HF), src2, dst2m)
    y3 = _tc_mid(s2, y2, dinv, b2.reshape(1, HH), W3)
    s3 = _mp_kernel(y3.reshape(NSC * NN, HF), src2, dst2m)
    h3 = _tc_post(s3, y3, dinv, b3.reshape(1, HH))

    sums, counts = _pool_kernel(h3, batch2)
    out = _tc_head(sums, counts.reshape(GG, 1), Wl, bl.reshape(1, OC))
    return out


# speedup vs baseline: 1.1226x; 1.1226x over previous
import functools

import jax
import jax.numpy as jnp
from jax import lax
from jax.experimental import pallas as pl
from jax.experimental.pallas import tpu as pltpu
from jax.experimental.pallas import tpu_sc as plsc

NN = 100000
EE = 1600000
HH = 32
HF = 16
GG = 256
OC = 45

NSC = 2
TPS = 16
CH = 80
GRP = 8
ECH = EE // CH
EG = ECH // GRP
DG = EG // NSC
MCH = 80
MGRP = 8
MECH = EE // MCH
MEG = MECH // MGRP
MGQ = MEG // TPS
NCH = NN // CH
NG = (NCH + GRP - 1) // GRP
ZR = 400
NZG = NN // ZR

_MESH = plsc.VectorSubcoreMesh(
    core_axis_name="c", subcore_axis_name="s", num_cores=NSC, num_subcores=TPS)

_f32 = jnp.float32
_i32 = jnp.int32


def _fill_rows(ref, nrows, value):
    v = jnp.full((16,), value, _f32)

    def body(i, _):
        ref[i, :] = v
        return 0

    lax.fori_loop(0, nrows, body, 0)


def _fill_flat(ref, n, value):
    v = jnp.full((16,), value, _f32)

    def body(i, _):
        ref[pl.ds(i * 16, 16)] = v
        return 0

    lax.fori_loop(0, n // 16, body, 0)


@functools.partial(
    pl.kernel,
    out_type=jax.ShapeDtypeStruct((NSC, NN), _f32),
    mesh=_MESH,
    compiler_params=pltpu.CompilerParams(use_tc_tiling_on_sc=False),
    scratch_types=[
        pltpu.VMEM((GRP, CH), _i32),
        pltpu.VMEM((CH,), _f32),
        pltpu.VMEM((ZR,), _f32),
        pltpu.VMEM_SHARED((NN,), _f32),
    ],
)
def _deg_kernel(dst_hbm, deg_out, dbuf, ones, zb, dacc):
    c = lax.axis_index("c")
    t = lax.axis_index("s")

    _fill_flat(zb, ZR, 0.0)
    _fill_flat(ones, CH, 1.0)

    def z_body(i, _):
        zg = t + TPS * i

        @pl.when(zg < NZG)
        def _():
            pltpu.sync_copy(zb, dacc.at[pl.ds(zg * ZR, ZR)])

        return 0

    lax.fori_loop(0, (NZG + TPS - 1) // TPS, z_body, 0)

    plsc.subcore_barrier()

    def blk(i, _):
        g = t + TPS * i

        @pl.when(g < DG)
        def _():
            pltpu.sync_copy(dst_hbm.at[pl.ds((c * DG + g) * GRP, GRP), :], dbuf)

            def inner(j, _):
                pltpu.sync_copy(ones, dacc.at[dbuf.at[j]], add=True)
                return 0

            lax.fori_loop(0, GRP, inner, 0)

        return 0

    lax.fori_loop(0, (DG + TPS - 1) // TPS, blk, 0)

    plsc.subcore_barrier()

    @pl.when(t == 0)
    def _():
        pltpu.sync_copy(dacc, deg_out.at[c])


@functools.partial(
    pl.kernel,
    out_type=jax.ShapeDtypeStruct((NSC, NN, HF), _f32),
    mesh=_MESH,
    compiler_params=pltpu.CompilerParams(use_tc_tiling_on_sc=False),
    scratch_types=[
        pltpu.VMEM((2, MGRP, MCH), _i32),
        pltpu.VMEM((4, MGRP, MCH), _i32),
        pltpu.VMEM((2, MGRP, MCH, HF), _f32),
        pltpu.VMEM((ZR, 16), _f32),
        pltpu.VMEM_SHARED((NN, HF), _f32),
        pltpu.SemaphoreType.DMA((2,)),
        pltpu.SemaphoreType.DMA((4,)),
        pltpu.SemaphoreType.DMA((2,)),
        pltpu.SemaphoreType.DMA((2,)),
    ],
)
def _mp_kernel(ys_hbm, src_hbm, dst_hbm, s_out, sbuf, dbuf, rows, zb, acc,
               issem, idsem, gsem, ssem):
    c = lax.axis_index("c")
    t = lax.axis_index("s")

    _fill_rows(zb, ZR, 0.0)

    def z_body(i, _):
        zg = t + TPS * i

        @pl.when(zg < NZG)
        def _():
            pltpu.sync_copy(zb, acc.at[pl.ds(zg * ZR, ZR), :])

        return 0

    lax.fori_loop(0, (NZG + TPS - 1) // TPS, z_body, 0)

    plsc.subcore_barrier()

    nx = MEG - TPS * MGQ
    gbase = t * MGQ + jnp.minimum(t, nx)
    mgt = jnp.where(t < nx, MGQ + 1, MGQ)

    def ld_src(i):
        g = gbase + i
        return pltpu.make_async_copy(
            src_hbm.at[c, pl.ds(g * MGRP, MGRP), :],
            sbuf.at[lax.rem(i, 2)], issem.at[lax.rem(i, 2)])

    def ld_dst(i):
        g = gbase + i
        return pltpu.make_async_copy(
            dst_hbm.at[pl.ds(g * MGRP, MGRP), :],
            dbuf.at[lax.rem(i, 4)], idsem.at[lax.rem(i, 4)])

    def gath(i, j):
        p = lax.rem(i, 2)
        return pltpu.make_async_copy(
            ys_hbm.at[sbuf.at[p, j]], rows.at[p, j], gsem.at[p])

    def scat(i, j):
        p = lax.rem(i, 2)
        return pltpu.make_async_copy(
            rows.at[p, j], acc.at[dbuf.at[lax.rem(i, 4), j]], ssem.at[p])

    ld_src(0).start()
    ld_dst(0).start()
    ld_src(1).start()
    ld_dst(1).start()
    ld_src(0).wait()

    def issue_gathers(i):
        def body(j, _):
            gath(i, j).start()
            return 0

        lax.fori_loop(0, MGRP, body, 0)

    def wait_scatters(i):
        def body(j, _):
            scat(i, j).wait()
            return 0

        lax.fori_loop(0, MGRP, body, 0)

    issue_gathers(0)

    def step(i, _):
        @pl.when(i >= 1)
        def _():
            wait_scatters(i - 1)

        ld_dst(i).wait()

        def chunk(j, _):
            gath(i, j).wait()
            scat(i, j).start(add=True)
            return 0

        lax.fori_loop(0, MGRP, chunk, 0)

        @pl.when(i + 2 < mgt)
        def _():
            ld_src(i + 2).start()
            ld_dst(i + 2).start()

        @pl.when(i + 1 < mgt)
        def _():
            ld_src(i + 1).wait()
            issue_gathers(i + 1)

        return 0

    lax.fori_loop(0, mgt, step, 0)
    wait_scatters(mgt - 1)

    plsc.subcore_barrier()

    def w_body(i, _):
        zg = t + TPS * i

        @pl.when(zg < NZG)
        def _():
            pltpu.sync_copy(acc.at[pl.ds(zg * ZR, ZR), :],
                            s_out.at[c, pl.ds(zg * ZR, ZR), :])

        return 0

    lax.fori_loop(0, (NZG + TPS - 1) // TPS, w_body, 0)


NB = 4000


def _tc_pre_body(x_ref, deg_ref, w_ref, y_ref, dinv_ref):
    d = deg_ref[0] + deg_ref[1] + 1.0
    dinv = lax.rsqrt(d)
    xw = jnp.dot(x_ref[...], w_ref[...], preferred_element_type=_f32)
    y = xw * dinv
    y_ref[0] = y[:, :HF]
    y_ref[1] = y[:, HF:]
    dinv_ref[...] = dinv


def _tc_mid_body(s_ref, y_ref, dinv_ref, b_ref, w_ref, y2_ref):
    dinv = dinv_ref[...]
    b = b_ref[...]
    h_lo = jnp.maximum(dinv * (s_ref[0] + y_ref[0]) + b[:, :HF], 0.0)
    h_hi = jnp.maximum(dinv * (s_ref[1] + y_ref[1]) + b[:, HF:], 0.0)
    w = w_ref[...]
    y2 = (jnp.dot(h_lo, w[:HF, :], preferred_element_type=_f32)
          + jnp.dot(h_hi, w[HF:, :], preferred_element_type=_f32)) * dinv
    y2_ref[0] = y2[:, :HF]
    y2_ref[1] = y2[:, HF:]


def _tc_tail_body(s_ref, y_ref, dinv_ref, b_ref, batch_ref, wl_ref, bl_ref,
                  out_ref, sums_ref, cnt_ref):
    i = pl.program_id(0)
    dinv = dinv_ref[...]
    b = b_ref[...]
    h_lo = dinv * (s_ref[0] + y_ref[0]) + b[:, :HF]
    h_hi = dinv * (s_ref[1] + y_ref[1]) + b[:, HF:]
    gids = lax.broadcasted_iota(_i32, (1, GG), 1)
    onehot = (batch_ref[...] == gids).astype(_f32)
    dn = (((0,), (0,)), ((), ()))
    slo = lax.dot_general(onehot, h_lo, dn,
                          preferred_element_type=_f32)
    shi = lax.dot_general(onehot, h_hi, dn, preferred_element_type=_f32)
    cnt = lax.dot_general(onehot, jnp.ones((NB, 1), _f32), dn,
                          preferred_element_type=_f32)

    @pl.when(i == 0)
    def _():
        sums_ref[0] = slo
        sums_ref[1] = shi
        cnt_ref[...] = cnt

    @pl.when(i > 0)
    def _():
        sums_ref[0] += slo
        sums_ref[1] += shi
        cnt_ref[...] += cnt

    @pl.when(i == _GRID - 1)
    def _():
        c = jnp.maximum(cnt_ref[...], 1.0)
        p_lo = sums_ref[0] / c
        p_hi = sums_ref[1] / c
        wl = wl_ref[...]
        out_ref[...] = (jnp.dot(p_lo, wl[:HF, :], preferred_element_type=_f32)
                        + jnp.dot(p_hi, wl[HF:, :],
                                  preferred_element_type=_f32)
                        + bl_ref[...])


_GRID = NN // NB

_spec_half = pl.BlockSpec((NSC, NB, HF), lambda i: (0, i, 0))
_spec_dinv = pl.BlockSpec((NB, 1), lambda i: (i, 0))


def _tc_pre(x, deg2, w1):
    return pl.pallas_call(
        _tc_pre_body,
        grid=(_GRID,),
        in_specs=[
            pl.BlockSpec((NB, 3), lambda i: (i, 0)),
            pl.BlockSpec((NSC, NB, 1), lambda i: (0, i, 0)),
            pl.BlockSpec((3, HH), lambda i: (0, 0)),
        ],
        out_specs=[_spec_half, _spec_dinv],
        out_shape=[jax.ShapeDtypeStruct((NSC, NN, HF), _f32),
                   jax.ShapeDtypeStruct((NN, 1), _f32)],
    )(x, deg2, w1)


def _tc_mid(s, y, dinv, b, w):
    return pl.pallas_call(
        _tc_mid_body,
        grid=(_GRID,),
        in_specs=[
            _spec_half, _spec_half, _spec_dinv,
            pl.BlockSpec((1, HH), lambda i: (0, 0)),
            pl.BlockSpec((HH, HH), lambda i: (0, 0)),
        ],
        out_specs=_spec_half,
        out_shape=jax.ShapeDtypeStruct((NSC, NN, HF), _f32),
    )(s, y, dinv, b, w)


def _tc_tail(s, y, dinv, b, batch1, wl, bl):
    return pl.pallas_call(
        _tc_tail_body,
        grid=(_GRID,),
        in_specs=[
            _spec_half, _spec_half, _spec_dinv,
            pl.BlockSpec((1, HH), lambda i: (0, 0)),
            pl.BlockSpec((NB, 1), lambda i: (i, 0)),
            pl.BlockSpec((HH, OC), lambda i: (0, 0)),
            pl.BlockSpec((1, OC), lambda i: (0, 0)),
        ],
        out_specs=pl.BlockSpec((GG, OC), lambda i: (0, 0)),
        out_shape=jax.ShapeDtypeStruct((GG, OC), _f32),
        scratch_shapes=[
            pltpu.VMEM((NSC, GG, HF), _f32),
            pltpu.VMEM((GG, 1), _f32),
        ],
    )(s, y, dinv, b, batch1, wl, bl)


def kernel(x, edge_index, batch, W1, b1, W2, b2, W3, b3, Wl, bl):
    src = edge_index[0]
    dst = edge_index[1]
    src2 = jnp.stack([src, src + NN]).reshape(NSC, MECH, MCH)
    dst2m = dst.reshape(MECH, MCH)
    dst2 = dst.reshape(ECH, CH)

    deg2 = _deg_kernel(dst2)
    deg2r = deg2.reshape(NSC, NN, 1)

    y1, dinv = _tc_pre(x, deg2r, W1)
    s1 = _mp_kernel(y1.reshape(NSC * NN, HF), src2, dst2m)
    y2 = _tc_mid(s1, y1, dinv, b1.reshape(1, HH), W2)
    s2 = _mp_kernel(y2.reshape(NSC * NN, HF), src2, dst2m)
    y3 = _tc_mid(s2, y2, dinv, b2.reshape(1, HH), W3)
    s3 = _mp_kernel(y3.reshape(NSC * NN, HF), src2, dst2m)
    out = _tc_tail(s3, y3, dinv, b3.reshape(1, HH), batch.reshape(NN, 1),
                   Wl, bl.reshape(1, OC))
    return out


# speedup vs baseline: 1.1585x; 1.0320x over previous
import functools

import jax
import jax.numpy as jnp
from jax import lax
from jax.experimental import pallas as pl
from jax.experimental.pallas import tpu as pltpu
from jax.experimental.pallas import tpu_sc as plsc

NN = 100000
EE = 1600000
HH = 32
HF = 16
GG = 256
OC = 45

NSC = 2
TPS = 16
CH = 80
GRP = 8
ECH = EE // CH
EG = ECH // GRP
DG = EG // NSC
MCH = 80
MGRP = 8
MECH = EE // MCH
MEG = MECH // MGRP
MGQ = MEG // TPS
NCH = NN // CH
NG = (NCH + GRP - 1) // GRP
ZR = 400
NZG = NN // ZR

_MESH = plsc.VectorSubcoreMesh(
    core_axis_name="c", subcore_axis_name="s", num_cores=NSC, num_subcores=TPS)

_f32 = jnp.float32
_i32 = jnp.int32


def _fill_rows(ref, nrows, value):
    v = jnp.full((16,), value, _f32)

    def body(i, _):
        ref[i, :] = v
        return 0

    lax.fori_loop(0, nrows, body, 0)


def _fill_flat(ref, n, value):
    v = jnp.full((16,), value, _f32)

    def body(i, _):
        ref[pl.ds(i * 16, 16)] = v
        return 0

    lax.fori_loop(0, n // 16, body, 0)


@functools.partial(
    pl.kernel,
    out_type=jax.ShapeDtypeStruct((NSC, NN), _f32),
    mesh=_MESH,
    compiler_params=pltpu.CompilerParams(use_tc_tiling_on_sc=False),
    scratch_types=[
        pltpu.VMEM((GRP, CH), _i32),
        pltpu.VMEM((CH,), _f32),
        pltpu.VMEM((ZR,), _f32),
        pltpu.VMEM_SHARED((NN,), _f32),
    ],
)
def _deg_kernel(ei_hbm, deg_out, dbuf, ones, zb, dacc):
    c = lax.axis_index("c")
    t = lax.axis_index("s")

    _fill_flat(zb, ZR, 0.0)
    _fill_flat(ones, CH, 1.0)

    def z_body(i, _):
        zg = t + TPS * i

        @pl.when(zg < NZG)
        def _():
            pltpu.sync_copy(zb, dacc.at[pl.ds(zg * ZR, ZR)])

        return 0

    lax.fori_loop(0, (NZG + TPS - 1) // TPS, z_body, 0)

    plsc.subcore_barrier()

    def blk(i, _):
        g = t + TPS * i

        @pl.when(g < DG)
        def _():
            pltpu.sync_copy(ei_hbm.at[1, pl.ds((c * DG + g) * GRP, GRP), :], dbuf)

            def inner(j, _):
                pltpu.sync_copy(ones, dacc.at[dbuf.at[j]], add=True)
                return 0

            lax.fori_loop(0, GRP, inner, 0)

        return 0

    lax.fori_loop(0, (DG + TPS - 1) // TPS, blk, 0)

    plsc.subcore_barrier()

    @pl.when(t == 0)
    def _():
        pltpu.sync_copy(dacc, deg_out.at[c])


@functools.partial(
    pl.kernel,
    out_type=jax.ShapeDtypeStruct((NSC, NN, HF), _f32),
    mesh=_MESH,
    compiler_params=pltpu.CompilerParams(use_tc_tiling_on_sc=False),
    scratch_types=[
        pltpu.VMEM((2, MGRP, MCH), _i32),
        pltpu.VMEM((4, MGRP, MCH), _i32),
        pltpu.VMEM((2, MGRP, MCH, HF), _f32),
        pltpu.VMEM((ZR, 16), _f32),
        pltpu.VMEM_SHARED((NN, HF), _f32),
        pltpu.SemaphoreType.DMA((2,)),
        pltpu.SemaphoreType.DMA((4,)),
        pltpu.SemaphoreType.DMA((2,)),
        pltpu.SemaphoreType.DMA((2,)),
    ],
)
def _mp_kernel(ys_hbm, ei_hbm, s_out, sbuf, dbuf, rows, zb, acc,
               issem, idsem, gsem, ssem):
    c = lax.axis_index("c")
    t = lax.axis_index("s")

    _fill_rows(zb, ZR, 0.0)

    def z_body(i, _):
        zg = t + TPS * i

        @pl.when(zg < NZG)
        def _():
            pltpu.sync_copy(zb, acc.at[pl.ds(zg * ZR, ZR), :])

        return 0

    lax.fori_loop(0, (NZG + TPS - 1) // TPS, z_body, 0)

    plsc.subcore_barrier()

    nx = MEG - TPS * MGQ
    gbase = t * MGQ + jnp.minimum(t, nx)
    mgt = jnp.where(t < nx, MGQ + 1, MGQ)

    def ld_src(i):
        g = gbase + i
        return pltpu.make_async_copy(
            ei_hbm.at[0, pl.ds(g * MGRP, MGRP), :],
            sbuf.at[lax.rem(i, 2)], issem.at[lax.rem(i, 2)])

    def ld_dst(i):
        g = gbase + i
        return pltpu.make_async_copy(
            ei_hbm.at[1, pl.ds(g * MGRP, MGRP), :],
            dbuf.at[lax.rem(i, 4)], idsem.at[lax.rem(i, 4)])

    def gath(i, j):
        p = lax.rem(i, 2)
        return pltpu.make_async_copy(
            ys_hbm.at[c].at[sbuf.at[p, j]], rows.at[p, j], gsem.at[p])

    def scat(i, j):
        p = lax.rem(i, 2)
        return pltpu.make_async_copy(
            rows.at[p, j], acc.at[dbuf.at[lax.rem(i, 4), j]], ssem.at[p])

    ld_src(0).start()
    ld_dst(0).start()
    ld_src(1).start()
    ld_dst(1).start()
    ld_src(0).wait()

    def issue_gathers(i):
        def body(j, _):
            gath(i, j).start()
            return 0

        lax.fori_loop(0, MGRP, body, 0)

    def wait_scatters(i):
        def body(j, _):
            scat(i, j).wait()
            return 0

        lax.fori_loop(0, MGRP, body, 0)

    issue_gathers(0)

    def step(i, _):
        @pl.when(i >= 1)
        def _():
            wait_scatters(i - 1)

        ld_dst(i).wait()

        def chunk(j, _):
            gath(i, j).wait()
            scat(i, j).start(add=True)
            return 0

        lax.fori_loop(0, MGRP, chunk, 0)

        @pl.when(i + 2 < mgt)
        def _():
            ld_src(i + 2).start()
            ld_dst(i + 2).start()

        @pl.when(i + 1 < mgt)
        def _():
            ld_src(i + 1).wait()
            issue_gathers(i + 1)

        return 0

    lax.fori_loop(0, mgt, step, 0)
    wait_scatters(mgt - 1)

    plsc.subcore_barrier()

    def w_body(i, _):
        zg = t + TPS * i

        @pl.when(zg < NZG)
        def _():
            pltpu.sync_copy(acc.at[pl.ds(zg * ZR, ZR), :],
                            s_out.at[c, pl.ds(zg * ZR, ZR), :])

        return 0

    lax.fori_loop(0, (NZG + TPS - 1) // TPS, w_body, 0)


NB = 4000


def _tc_pre_body(x_ref, deg_ref, w_ref, y_ref, dinv_ref):
    d = deg_ref[0] + deg_ref[1] + 1.0
    dinv = lax.rsqrt(d)
    xw = jnp.dot(x_ref[...], w_ref[...], preferred_element_type=_f32)
    y = xw * dinv
    y_ref[0] = y[:, :HF]
    y_ref[1] = y[:, HF:]
    dinv_ref[...] = dinv


def _tc_mid_body(s_ref, y_ref, dinv_ref, b_ref, w_ref, y2_ref):
    dinv = dinv_ref[...]
    b = b_ref[...]
    h_lo = jnp.maximum(dinv * (s_ref[0] + y_ref[0]) + b[:, :HF], 0.0)
    h_hi = jnp.maximum(dinv * (s_ref[1] + y_ref[1]) + b[:, HF:], 0.0)
    w = w_ref[...]
    y2 = (jnp.dot(h_lo, w[:HF, :], preferred_element_type=_f32)
          + jnp.dot(h_hi, w[HF:, :], preferred_element_type=_f32)) * dinv
    y2_ref[0] = y2[:, :HF]
    y2_ref[1] = y2[:, HF:]


def _tc_tail_body(s_ref, y_ref, dinv_ref, b_ref, batch_ref, wl_ref, bl_ref,
                  out_ref, sums_ref, cnt_ref):
    i = pl.program_id(0)
    dinv = dinv_ref[...]
    b = b_ref[...]
    h_lo = dinv * (s_ref[0] + y_ref[0]) + b[:, :HF]
    h_hi = dinv * (s_ref[1] + y_ref[1]) + b[:, HF:]
    gids = lax.broadcasted_iota(_i32, (1, GG), 1)
    onehot = (batch_ref[...] == gids).astype(_f32)
    dn = (((0,), (0,)), ((), ()))
    slo = lax.dot_general(onehot, h_lo, dn,
                          preferred_element_type=_f32)
    shi = lax.dot_general(onehot, h_hi, dn, preferred_element_type=_f32)
    cnt = lax.dot_general(onehot, jnp.ones((NB, 1), _f32), dn,
                          preferred_element_type=_f32)

    @pl.when(i == 0)
    def _():
        sums_ref[0] = slo
        sums_ref[1] = shi
        cnt_ref[...] = cnt

    @pl.when(i > 0)
    def _():
        sums_ref[0] += slo
        sums_ref[1] += shi
        cnt_ref[...] += cnt

    @pl.when(i == _GRID - 1)
    def _():
        c = jnp.maximum(cnt_ref[...], 1.0)
        p_lo = sums_ref[0] / c
        p_hi = sums_ref[1] / c
        wl = wl_ref[...]
        out_ref[...] = (jnp.dot(p_lo, wl[:HF, :], preferred_element_type=_f32)
                        + jnp.dot(p_hi, wl[HF:, :],
                                  preferred_element_type=_f32)
                        + bl_ref[...])


_GRID = NN // NB

_spec_half = pl.BlockSpec((NSC, NB, HF), lambda i: (0, i, 0))
_spec_dinv = pl.BlockSpec((NB, 1), lambda i: (i, 0))


def _tc_pre(x, deg2, w1):
    return pl.pallas_call(
        _tc_pre_body,
        grid=(_GRID,),
        in_specs=[
            pl.BlockSpec((NB, 3), lambda i: (i, 0)),
            pl.BlockSpec((NSC, NB, 1), lambda i: (0, i, 0)),
            pl.BlockSpec((3, HH), lambda i: (0, 0)),
        ],
        out_specs=[_spec_half, _spec_dinv],
        out_shape=[jax.ShapeDtypeStruct((NSC, NN, HF), _f32),
                   jax.ShapeDtypeStruct((NN, 1), _f32)],
    )(x, deg2, w1)


def _tc_mid(s, y, dinv, b, w):
    return pl.pallas_call(
        _tc_mid_body,
        grid=(_GRID,),
        in_specs=[
            _spec_half, _spec_half, _spec_dinv,
            pl.BlockSpec((1, HH), lambda i: (0, 0)),
            pl.BlockSpec((HH, HH), lambda i: (0, 0)),
        ],
        out_specs=_spec_half,
        out_shape=jax.ShapeDtypeStruct((NSC, NN, HF), _f32),
    )(s, y, dinv, b, w)


def _tc_tail(s, y, dinv, b, batch1, wl, bl):
    return pl.pallas_call(
        _tc_tail_body,
        grid=(_GRID,),
        in_specs=[
            _spec_half, _spec_half, _spec_dinv,
            pl.BlockSpec((1, HH), lambda i: (0, 0)),
            pl.BlockSpec((NB, 1), lambda i: (i, 0)),
            pl.BlockSpec((HH, OC), lambda i: (0, 0)),
            pl.BlockSpec((1, OC), lambda i: (0, 0)),
        ],
        out_specs=pl.BlockSpec((GG, OC), lambda i: (0, 0)),
        out_shape=jax.ShapeDtypeStruct((GG, OC), _f32),
        scratch_shapes=[
            pltpu.VMEM((NSC, GG, HF), _f32),
            pltpu.VMEM((GG, 1), _f32),
        ],
    )(s, y, dinv, b, batch1, wl, bl)


def kernel(x, edge_index, batch, W1, b1, W2, b2, W3, b3, Wl, bl):
    ei3 = edge_index.reshape(2, ECH, CH)

    deg2 = _deg_kernel(ei3)
    deg2r = deg2.reshape(NSC, NN, 1)

    y1, dinv = _tc_pre(x, deg2r, W1)
    s1 = _mp_kernel(y1, ei3)
    y2 = _tc_mid(s1, y1, dinv, b1.reshape(1, HH), W2)
    s2 = _mp_kernel(y2, ei3)
    y3 = _tc_mid(s2, y2, dinv, b2.reshape(1, HH), W3)
    s3 = _mp_kernel(y3, ei3)
    out = _tc_tail(s3, y3, dinv, b3.reshape(1, HH), batch.reshape(NN, 1),
                   Wl, bl.reshape(1, OC))
    return out
